# baseline (speedup 1.0000x reference)
import functools

import jax
import jax.numpy as jnp
from jax import lax
from jax.experimental import pallas as pl
from jax.experimental.pallas import tpu as pltpu
from jax.experimental.pallas import tpu_sc as plsc

F32 = jnp.float32
I32 = jnp.int32

NC = 2
NS = 16
LN = 16
NW = NC * NS

H = 8
C = 64
HC = H * C
NCHUNK = 4
CW = HC // NCHUNK


def _mm_attn_body(x_ref, w_ref, am_ref, h_ref, al_ref):
    h = jnp.dot(x_ref[...], w_ref[...], preferred_element_type=F32)
    h_ref[...] = h
    al_ref[...] = jnp.dot(h, am_ref[...], preferred_element_type=F32)


def _mm_attn(x, W, AM, bn=400):
    n, din = x.shape
    hc = W.shape[1]
    return pl.pallas_call(
        _mm_attn_body,
        grid=(n // bn,),
        in_specs=[
            pl.BlockSpec((bn, din), lambda i: (i, 0)),
            pl.BlockSpec((din, hc), lambda i: (0, 0)),
            pl.BlockSpec((hc, 128), lambda i: (0, 0)),
        ],
        out_specs=[
            pl.BlockSpec((bn, hc), lambda i: (i, 0)),
            pl.BlockSpec((bn, 128), lambda i: (i, 0)),
        ],
        out_shape=[
            jax.ShapeDtypeStruct((n, hc), F32),
            jax.ShapeDtypeStruct((n, 128), F32),
        ],
    )(x, W, AM)


def _norm_elu_body(agg_ref, d0_ref, d1_ref, e8_ref, b_ref, out_ref):
    den = d0_ref[...] + d1_ref[...]
    dexp = jnp.dot(den, e8_ref[...], preferred_element_type=F32)
    z = agg_ref[...] / (dexp + 1e-16) + b_ref[...]
    out_ref[...] = jnp.where(z > 0, z, jnp.exp(jnp.minimum(z, 0.0)) - 1.0)


def _norm_elu(agg, d0, d1, e8, b2d, bn=400):
    n = agg.shape[0]
    return pl.pallas_call(
        _norm_elu_body,
        grid=(n // bn,),
        in_specs=[
            pl.BlockSpec((bn, HC), lambda i: (i, 0)),
            pl.BlockSpec((bn, H), lambda i: (i, 0)),
            pl.BlockSpec((bn, H), lambda i: (i, 0)),
            pl.BlockSpec((H, HC), lambda i: (0, 0)),
            pl.BlockSpec((1, HC), lambda i: (0, 0)),
        ],
        out_specs=pl.BlockSpec((bn, HC), lambda i: (i, 0)),
        out_shape=jax.ShapeDtypeStruct((n, HC), F32),
    )(agg, d0, d1, e8, b2d)


def _pool_body(x_ref, b_ref, ps_ref, cnt_ref):
    i = pl.program_id(0)

    @pl.when(i == 0)
    def _():
        ps_ref[...] = jnp.zeros_like(ps_ref)
        cnt_ref[...] = jnp.zeros_like(cnt_ref)

    bn = x_ref.shape[0]
    g = ps_ref.shape[0]
    bb = jnp.broadcast_to(b_ref[...].reshape(1, bn), (g, bn))
    gi = lax.broadcasted_iota(I32, (g, bn), 0)
    p = (bb == gi).astype(F32)
    ps_ref[...] += jnp.dot(p, x_ref[...], preferred_element_type=F32)
    cnt_ref[...] += jnp.broadcast_to(
        jnp.sum(p, axis=1, keepdims=True), cnt_ref.shape)


def _pool(x, batch3, g, bn=400):
    n = x.shape[0]
    return pl.pallas_call(
        _pool_body,
        grid=(n // bn,),
        in_specs=[
            pl.BlockSpec((bn, HC), lambda i: (i, 0)),
            pl.BlockSpec((1, 1, bn), lambda i: (i, 0, 0)),
        ],
        out_specs=[
            pl.BlockSpec((g, HC), lambda i: (0, 0)),
            pl.BlockSpec((g, 128), lambda i: (0, 0)),
        ],
        out_shape=[
            jax.ShapeDtypeStruct((g, HC), F32),
            jax.ShapeDtypeStruct((g, 128), F32),
        ],
    )(x, batch3)


def _head_body(ps_ref, cnt_ref, w_ref, b_ref, out_ref):
    cnt = jnp.maximum(cnt_ref[:, 0:1], 1.0)
    pooled = ps_ref[...] / cnt
    logits = jnp.dot(pooled, w_ref[...], preferred_element_type=F32) + b_ref[...]
    mask = lax.broadcasted_iota(I32, logits.shape, 1) < 10
    logits = jnp.where(mask, logits, -1e30)
    m = jnp.max(logits, axis=1, keepdims=True)
    lse = m + jnp.log(jnp.sum(jnp.exp(logits - m), axis=1, keepdims=True))
    out_ref[...] = logits - lse


def _head(ps, cnt, wp, bp, g):
    return pl.pallas_call(
        _head_body,
        grid=(1,),
        in_specs=[
            pl.BlockSpec((g, HC), lambda i: (0, 0)),
            pl.BlockSpec((g, 128), lambda i: (0, 0)),
            pl.BlockSpec((HC, 128), lambda i: (0, 0)),
            pl.BlockSpec((1, 128), lambda i: (0, 0)),
        ],
        out_specs=pl.BlockSpec((g, 128), lambda i: (0, 0)),
        out_shape=jax.ShapeDtypeStruct((g, 128), F32),
    )(ps, cnt, wp, bp)


_GDN = lax.GatherDimensionNumbers(
    offset_dims=(), collapsed_slice_dims=(0,), start_index_map=(0,))


def _vperm(v, idx):
    return lax.gather(v, idx[:, None], _GDN, (1,),
                      mode=lax.GatherScatterMode.PROMISE_IN_BOUNDS)

def _make_sc_edge_w(n, e):
    eb = e // NW
    ba = 80
    nblk = eb // ba
    nfl = n * H
    st = nfl // NS
    mesh = plsc.VectorSubcoreMesh(core_axis_name="c", subcore_axis_name="s")

    @functools.partial(
        pl.kernel,
        out_type=[
            jax.ShapeDtypeStruct((e * H,), F32),
            jax.ShapeDtypeStruct((2 * nfl,), F32),
        ],
        mesh=mesh,
        scratch_types=[
            pltpu.VMEM((ba,), I32),
            pltpu.VMEM((ba,), I32),
            pltpu.VMEM((ba, 128), F32),
            pltpu.VMEM((ba, 128), F32),
            pltpu.VMEM((ba * H,), F32),
            pltpu.VMEM((ba * H // 128, 128), I32),
            pltpu.VMEM((st + 16, ), F32),
            pltpu.VMEM_SHARED((nfl,), F32),
            pltpu.SemaphoreType.DMA,
            pltpu.SemaphoreType.DMA,
        ],
    )
    def kern(al_hbm, src_hbm, dst_hbm, w_hbm, denp_hbm,
             srcb, dstb, asb, adb, wfl, idx2, zb, den_sh, sem1, sem2):
        c = lax.axis_index("c")
        s = lax.axis_index("s")
        wid = c * NS + s
        iota = lax.iota(I32, LN)
        zv = iota.astype(F32) * 0.0
        i8 = iota & 7
        hi = iota >> 3
        lo8 = iota < 8

        def zrow(i, car):
            zb[pl.ds(i * LN, LN)] = zv
            return car
        lax.fori_loop(0, (st + 16) // LN, zrow, 0)
        pltpu.sync_copy(zb.at[pl.ds(0, st)], den_sh.at[pl.ds(s * st, st)])
        plsc.subcore_barrier()

        def blk(b_i, car):
            ebase = wid * eb + b_i * ba
            pltpu.sync_copy(src_hbm.at[pl.ds(ebase, ba)], srcb)
            pltpu.sync_copy(dst_hbm.at[pl.ds(ebase, ba)], dstb)
            cp1 = pltpu.async_copy(al_hbm.at[srcb], asb, sem1)
            cp2 = pltpu.async_copy(al_hbm.at[dstb], adb, sem2)
            cp1.wait()
            cp2.wait()

            @plsc.parallel_loop(0, ba // 2, unroll=4)
            def pair(p):
                e0 = 2 * p
                a0 = _vperm(asb[e0, pl.ds(0, LN)], i8)
                a1 = _vperm(asb[e0 + 1, pl.ds(0, LN)], i8)
                ga = jnp.where(lo8, a0, a1)
                d0 = _vperm(adb[e0, pl.ds(0, LN)], i8 + 8)
                d1 = _vperm(adb[e0 + 1, pl.ds(0, LN)], i8 + 8)
                gd = jnp.where(lo8, d0, d1)
                sv = ga + gd
                lv = jnp.maximum(sv, sv * 0.2)
                wfl[pl.ds(p * LN, LN)] = jnp.exp(lv)

            @plsc.parallel_loop(0, ba // LN, unroll=2)
            def idxgrp(i):
                dv = dstb[pl.ds(i * LN, LN)]
                for j in range(8):
                    idxv = _vperm(dv, 2 * j + hi) * H + i8
                    idx2[i, pl.ds(j * LN, LN)] = idxv

            pltpu.sync_copy(wfl, w_hbm.at[pl.ds(ebase * H, ba * H)])
            for q in range(ba * H // 128):
                pltpu.sync_copy(wfl.at[pl.ds(q * 128, 128)],
                                den_sh.at[idx2.at[q]], add=True)
            return car
        lax.fori_loop(0, nblk, blk, 0)

        plsc.subcore_barrier()
        pltpu.sync_copy(den_sh.at[pl.ds(s * st, st)], zb.at[pl.ds(0, st)])
        pltpu.sync_copy(zb.at[pl.ds(0, st)],
                        denp_hbm.at[pl.ds(c * nfl + s * st, st)])

    return kern


def _make_sc_agg(n, e):
    eb = e // NS
    bb = 80
    nblk = eb // bb
    rpt = 624
    tail = n - rpt * NS
    zr = 48
    mesh = plsc.VectorSubcoreMesh(core_axis_name="c", subcore_axis_name="s")

    @functools.partial(
        pl.kernel,
        out_type=jax.ShapeDtypeStruct((NCHUNK * n, CW), F32),
        mesh=mesh,
        scratch_types=[
            pltpu.VMEM((bb,), I32),
            pltpu.VMEM((bb,), I32),
            pltpu.VMEM((bb,), I32),
            pltpu.VMEM((bb,), I32),
            pltpu.VMEM((1, bb), I32),
            pltpu.VMEM((1, bb), I32),
            pltpu.VMEM((bb,), I32),
            pltpu.VMEM((bb,), I32),
            pltpu.VMEM((bb * H,), F32),
            pltpu.VMEM((bb * H,), F32),
            pltpu.VMEM((bb, CW), F32),
            pltpu.VMEM((bb, CW), F32),
            pltpu.VMEM((bb, CW), F32),
            pltpu.VMEM((zr, CW), F32),
            pltpu.VMEM_SHARED((n, CW), F32),
            pltpu.SemaphoreType.DMA,
            pltpu.SemaphoreType.DMA,
        ],
    )
    def kern(h4_hbm, w_hbm, src_hbm, dst_hbm, agg_hbm,
             srcb0, srcb1, dstb0, dstb1, dst2a, dst2b, offb0, offb1,
             wfl0, wfl1, hr0, hr1, msg, zb, acc_sh, semA, semB):
        c = lax.axis_index("c")
        s = lax.axis_index("s")
        iota = lax.iota(I32, LN)
        zv = iota.astype(F32) * 0.0
        srcb = [srcb0, srcb1]
        dstb = [dstb0, dstb1]
        dst2 = [dst2a, dst2b]
        offb = [offb0, offb1]
        wfl = [wfl0, wfl1]
        hr = [hr0, hr1]
        sem = [semA, semB]

        def zrow(i, car):
            zb[i // (CW // LN), pl.ds((i % (CW // LN)) * LN, LN)] = zv
            return car
        lax.fori_loop(0, zr * CW // LN, zrow, 0)

        for k in range(NCHUNK // NC):
            cc = c * (NCHUNK // NC) + k
            ccn = cc * n
            h2 = 2 * cc
            for q in range(rpt // zr):
                pltpu.sync_copy(zb, acc_sh.at[pl.ds(s * rpt + q * zr, zr)])

            @pl.when(s == NS - 1)
            def _():
                pltpu.sync_copy(zb.at[pl.ds(0, tail)],
                                acc_sh.at[pl.ds(rpt * NS, tail)])
            plsc.subcore_barrier()

            def issue(b_i, bsel):
                ebase = s * eb + b_i * bb
                pltpu.sync_copy(src_hbm.at[pl.ds(ebase, bb)], srcb[bsel])

                @plsc.parallel_loop(0, bb // LN, unroll=2)
                def _(i):
                    offb[bsel][pl.ds(i * LN, LN)] = \
                        srcb[bsel][pl.ds(i * LN, LN)] + ccn
                pltpu.async_copy(h4_hbm.at[offb[bsel]], hr[bsel], sem[bsel])
                pltpu.sync_copy(dst_hbm.at[pl.ds(ebase, bb)], dstb[bsel])
                for i in range(bb // LN):
                    dst2[bsel][0, pl.ds(i * LN, LN)] = \
                        dstb[bsel][pl.ds(i * LN, LN)]
                pltpu.sync_copy(w_hbm.at[pl.ds(ebase * H, bb * H)], wfl[bsel])

            def process(b_i, bsel):
                pltpu.make_async_copy(
                    h4_hbm.at[offb[bsel]], hr[bsel], sem[bsel]).wait()

                @plsc.parallel_loop(0, bb // 2, unroll=8)
                def _(p):
                    wv = wfl[bsel][pl.ds(p * LN, LN)]
                    e0 = 2 * p
                    for t in range(2):
                        w0 = _vperm(wv, jnp.full((LN,), 8 * t + h2, I32))
                        w1 = _vperm(wv, jnp.full((LN,), 8 * t + h2 + 1, I32))
                        for j in range(CW // LN):
                            wvv = w0 if j < (CW // LN // 2) else w1
                            hv = hr[bsel][e0 + t, pl.ds(j * LN, LN)]
                            msg[e0 + t, pl.ds(j * LN, LN)] = hv * wvv
                pltpu.sync_copy(msg, acc_sh.at[dst2[bsel].at[0]],
                                add=True)

            issue(0, 0)

            def g_body(g, car):
                b0 = 2 * g
                issue(b0 + 1, 1)
                process(b0, 0)

                @pl.when(b0 + 2 < nblk)
                def _():
                    issue(b0 + 2, 0)
                process(b0 + 1, 1)
                return car
            lax.fori_loop(0, nblk // 2, g_body, 0)

            plsc.subcore_barrier()
            pltpu.sync_copy(acc_sh.at[pl.ds(s * rpt, rpt)],
                            agg_hbm.at[pl.ds(cc * n + s * rpt, rpt)])

            @pl.when(s == NS - 1)
            def _():
                pltpu.sync_copy(acc_sh.at[pl.ds(rpt * NS, tail)],
                                agg_hbm.at[pl.ds(cc * n + rpt * NS, tail)])
            plsc.subcore_barrier()

    return kern


def _attn_mat(a_src, a_dst):
    eye = jnp.eye(H, dtype=F32)
    ms = (a_src[:, :, None] * eye[:, None, :]).reshape(HC, H)
    md = (a_dst[:, :, None] * eye[:, None, :]).reshape(HC, H)
    return jnp.pad(jnp.concatenate([ms, md], axis=1), ((0, 0), (0, 112)))


def _to_chunk_major(h, n):
    return h.reshape(n, NCHUNK, CW).transpose(1, 0, 2).reshape(NCHUNK * n, CW)


def _from_chunk_major(a, n):
    return a.reshape(NCHUNK, n, CW).transpose(1, 0, 2).reshape(n, HC)


def kernel(x, edge_index, edge_attr, batch, W1, a_src1, a_dst1, b1,
           W2, a_src2, a_dst2, b2, W3, a_src3, a_dst3, b3, linW, linb):
    n, _ = x.shape
    e = edge_index.shape[1]
    g = 16
    src = edge_index[0].astype(I32)
    dst = edge_index[1].astype(I32)

    sc_w = _make_sc_edge_w(n, e)
    sc_agg = _make_sc_agg(n, e)
    e8 = jnp.repeat(jnp.eye(H, dtype=F32), C, axis=1)

    def layer(xin, W, a_s, a_d, b):
        h, al = _mm_attn(xin, W, _attn_mat(a_s, a_d))
        we, denp = sc_w(al, src, dst)
        aggf = sc_agg(_to_chunk_major(h, n), we, src, dst)
        agg = _from_chunk_major(aggf, n)
        dp = denp.reshape(2, n, H)
        return _norm_elu(agg, dp[0], dp[1], e8, b.reshape(1, HC))

    x2 = layer(x, W1, a_src1, a_dst1, b1)
    x3 = layer(x2, W2, a_src2, a_dst2, b2)
    x4 = layer(x3, W3, a_src3, a_dst3, b3)

    batch3 = batch.astype(I32).reshape(n // 400, 1, 400)
    ps, cnt = _pool(x4, batch3, g)
    wp = jnp.pad(linW, ((0, 0), (0, 128 - linW.shape[1])))
    bp = jnp.pad(linb, (0, 128 - linb.shape[0])).reshape(1, 128)
    out = _head(ps, cnt, wp, bp, g)
    return out[:, :linW.shape[1]]


# speedup vs baseline: 1.4135x; 1.4135x over previous
import functools

import jax
import jax.numpy as jnp
from jax import lax
from jax.experimental import pallas as pl
from jax.experimental.pallas import tpu as pltpu
from jax.experimental.pallas import tpu_sc as plsc

F32 = jnp.float32
I32 = jnp.int32

NC = 2
NS = 16
LN = 16
NW = NC * NS

H = 8
C = 64
HC = H * C
NCHUNK = 4
CW = HC // NCHUNK


def _mm_attn_body(x_ref, w_ref, am_ref, h_ref, al_ref):
    h = jnp.dot(x_ref[...], w_ref[...], preferred_element_type=F32)
    h_ref[...] = h
    al_ref[...] = jnp.dot(h, am_ref[...], preferred_element_type=F32)


def _mm_attn(x, W, AM, bn=400):
    n, din = x.shape
    hc = W.shape[1]
    return pl.pallas_call(
        _mm_attn_body,
        grid=(n // bn,),
        in_specs=[
            pl.BlockSpec((bn, din), lambda i: (i, 0)),
            pl.BlockSpec((din, hc), lambda i: (0, 0)),
            pl.BlockSpec((hc, 128), lambda i: (0, 0)),
        ],
        out_specs=[
            pl.BlockSpec((bn, hc), lambda i: (i, 0)),
            pl.BlockSpec((bn, 128), lambda i: (i, 0)),
        ],
        out_shape=[
            jax.ShapeDtypeStruct((n, hc), F32),
            jax.ShapeDtypeStruct((n, 128), F32),
        ],
    )(x, W, AM)


def _norm_elu_body(agg_ref, d0_ref, d1_ref, e8_ref, b_ref, out_ref):
    den = d0_ref[...] + d1_ref[...]
    dexp = jnp.dot(den, e8_ref[...], preferred_element_type=F32)
    z = agg_ref[...] / (dexp + 1e-16) + b_ref[...]
    out_ref[...] = jnp.where(z > 0, z, jnp.exp(jnp.minimum(z, 0.0)) - 1.0)


def _norm_elu(agg, d0, d1, e8, b2d, bn=400):
    n = agg.shape[0]
    return pl.pallas_call(
        _norm_elu_body,
        grid=(n // bn,),
        in_specs=[
            pl.BlockSpec((bn, HC), lambda i: (i, 0)),
            pl.BlockSpec((bn, H), lambda i: (i, 0)),
            pl.BlockSpec((bn, H), lambda i: (i, 0)),
            pl.BlockSpec((H, HC), lambda i: (0, 0)),
            pl.BlockSpec((1, HC), lambda i: (0, 0)),
        ],
        out_specs=pl.BlockSpec((bn, HC), lambda i: (i, 0)),
        out_shape=jax.ShapeDtypeStruct((n, HC), F32),
    )(agg, d0, d1, e8, b2d)


def _pool_body(x_ref, b_ref, ps_ref, cnt_ref):
    i = pl.program_id(0)

    @pl.when(i == 0)
    def _():
        ps_ref[...] = jnp.zeros_like(ps_ref)
        cnt_ref[...] = jnp.zeros_like(cnt_ref)

    bn = x_ref.shape[0]
    g = ps_ref.shape[0]
    bb = jnp.broadcast_to(b_ref[...].reshape(1, bn), (g, bn))
    gi = lax.broadcasted_iota(I32, (g, bn), 0)
    p = (bb == gi).astype(F32)
    ps_ref[...] += jnp.dot(p, x_ref[...], preferred_element_type=F32)
    cnt_ref[...] += jnp.broadcast_to(
        jnp.sum(p, axis=1, keepdims=True), cnt_ref.shape)


def _pool(x, batch3, g, bn=400):
    n = x.shape[0]
    return pl.pallas_call(
        _pool_body,
        grid=(n // bn,),
        in_specs=[
            pl.BlockSpec((bn, HC), lambda i: (i, 0)),
            pl.BlockSpec((1, 1, bn), lambda i: (i, 0, 0)),
        ],
        out_specs=[
            pl.BlockSpec((g, HC), lambda i: (0, 0)),
            pl.BlockSpec((g, 128), lambda i: (0, 0)),
        ],
        out_shape=[
            jax.ShapeDtypeStruct((g, HC), F32),
            jax.ShapeDtypeStruct((g, 128), F32),
        ],
    )(x, batch3)


def _head_body(ps_ref, cnt_ref, w_ref, b_ref, out_ref):
    cnt = jnp.maximum(cnt_ref[:, 0:1], 1.0)
    pooled = ps_ref[...] / cnt
    logits = jnp.dot(pooled, w_ref[...], preferred_element_type=F32) + b_ref[...]
    mask = lax.broadcasted_iota(I32, logits.shape, 1) < 10
    logits = jnp.where(mask, logits, -1e30)
    m = jnp.max(logits, axis=1, keepdims=True)
    lse = m + jnp.log(jnp.sum(jnp.exp(logits - m), axis=1, keepdims=True))
    out_ref[...] = logits - lse


def _head(ps, cnt, wp, bp, g):
    return pl.pallas_call(
        _head_body,
        grid=(1,),
        in_specs=[
            pl.BlockSpec((g, HC), lambda i: (0, 0)),
            pl.BlockSpec((g, 128), lambda i: (0, 0)),
            pl.BlockSpec((HC, 128), lambda i: (0, 0)),
            pl.BlockSpec((1, 128), lambda i: (0, 0)),
        ],
        out_specs=pl.BlockSpec((g, 128), lambda i: (0, 0)),
        out_shape=jax.ShapeDtypeStruct((g, 128), F32),
    )(ps, cnt, wp, bp)


_GDN = lax.GatherDimensionNumbers(
    offset_dims=(), collapsed_slice_dims=(0,), start_index_map=(0,))


def _vperm(v, idx):
    return lax.gather(v, idx[:, None], _GDN, (1,),
                      mode=lax.GatherScatterMode.PROMISE_IN_BOUNDS)

def _make_sc_edge_w(n, e):
    eb = e // NW
    ba = 80
    nblk = eb // ba
    nfl = n * H
    st = nfl // NS
    mesh = plsc.VectorSubcoreMesh(core_axis_name="c", subcore_axis_name="s")

    @functools.partial(
        pl.kernel,
        out_type=[
            jax.ShapeDtypeStruct((e * H,), F32),
            jax.ShapeDtypeStruct((2 * nfl,), F32),
        ],
        mesh=mesh,
        scratch_types=[
            pltpu.VMEM((ba,), I32),
            pltpu.VMEM((ba,), I32),
            pltpu.VMEM((ba, 128), F32),
            pltpu.VMEM((ba, 128), F32),
            pltpu.VMEM((ba * H,), F32),
            pltpu.VMEM((ba * H // 128, 128), I32),
            pltpu.VMEM((st + 16, ), F32),
            pltpu.VMEM_SHARED((nfl,), F32),
            pltpu.SemaphoreType.DMA,
            pltpu.SemaphoreType.DMA,
        ],
    )
    def kern(al_hbm, src_hbm, dst_hbm, w_hbm, denp_hbm,
             srcb, dstb, asb, adb, wfl, idx2, zb, den_sh, sem1, sem2):
        c = lax.axis_index("c")
        s = lax.axis_index("s")
        wid = c * NS + s
        iota = lax.iota(I32, LN)
        zv = iota.astype(F32) * 0.0
        i8 = iota & 7
        hi = iota >> 3
        lo8 = iota < 8

        def zrow(i, car):
            zb[pl.ds(i * LN, LN)] = zv
            return car
        lax.fori_loop(0, (st + 16) // LN, zrow, 0)
        pltpu.sync_copy(zb.at[pl.ds(0, st)], den_sh.at[pl.ds(s * st, st)])
        plsc.subcore_barrier()

        def blk(b_i, car):
            ebase = wid * eb + b_i * ba
            pltpu.sync_copy(src_hbm.at[pl.ds(ebase, ba)], srcb)
            pltpu.sync_copy(dst_hbm.at[pl.ds(ebase, ba)], dstb)
            cp1 = pltpu.async_copy(al_hbm.at[srcb], asb, sem1)
            cp2 = pltpu.async_copy(al_hbm.at[dstb], adb, sem2)
            cp1.wait()
            cp2.wait()

            @plsc.parallel_loop(0, ba // 2, unroll=4)
            def pair(p):
                e0 = 2 * p
                a0 = _vperm(asb[e0, pl.ds(0, LN)], i8)
                a1 = _vperm(asb[e0 + 1, pl.ds(0, LN)], i8)
                ga = jnp.where(lo8, a0, a1)
                d0 = _vperm(adb[e0, pl.ds(0, LN)], i8 + 8)
                d1 = _vperm(adb[e0 + 1, pl.ds(0, LN)], i8 + 8)
                gd = jnp.where(lo8, d0, d1)
                sv = ga + gd
                lv = jnp.maximum(sv, sv * 0.2)
                wfl[pl.ds(p * LN, LN)] = jnp.exp(lv)

            @plsc.parallel_loop(0, ba // LN, unroll=2)
            def idxgrp(i):
                dv = dstb[pl.ds(i * LN, LN)]
                for j in range(8):
                    idxv = _vperm(dv, 2 * j + hi) * H + i8
                    idx2[i, pl.ds(j * LN, LN)] = idxv

            pltpu.sync_copy(wfl, w_hbm.at[pl.ds(ebase * H, ba * H)])
            for q in range(ba * H // 128):
                pltpu.sync_copy(wfl.at[pl.ds(q * 128, 128)],
                                den_sh.at[idx2.at[q]], add=True)
            return car
        lax.fori_loop(0, nblk, blk, 0)

        plsc.subcore_barrier()
        pltpu.sync_copy(den_sh.at[pl.ds(s * st, st)], zb.at[pl.ds(0, st)])
        pltpu.sync_copy(zb.at[pl.ds(0, st)],
                        denp_hbm.at[pl.ds(c * nfl + s * st, st)])

    return kern


def _make_sc_agg(n, e):
    eb = e // NS
    bb = 80
    nblk = eb // bb
    rpt = 624
    tail = n - rpt * NS
    zr = 48
    mesh = plsc.VectorSubcoreMesh(core_axis_name="c", subcore_axis_name="s")

    @functools.partial(
        pl.kernel,
        out_type=jax.ShapeDtypeStruct((NCHUNK * n, CW), F32),
        mesh=mesh,
        scratch_types=[
            pltpu.VMEM((bb,), I32),
            pltpu.VMEM((bb,), I32),
            pltpu.VMEM((bb,), I32),
            pltpu.VMEM((bb,), I32),
            pltpu.VMEM((1, bb), I32),
            pltpu.VMEM((1, bb), I32),
            pltpu.VMEM((bb,), I32),
            pltpu.VMEM((bb,), I32),
            pltpu.VMEM((bb * H,), F32),
            pltpu.VMEM((bb * H,), F32),
            pltpu.VMEM((bb, CW), F32),
            pltpu.VMEM((bb, CW), F32),
            pltpu.VMEM((bb, CW), F32),
            pltpu.VMEM((zr, CW), F32),
            pltpu.VMEM_SHARED((n, CW), F32),
            pltpu.SemaphoreType.DMA,
            pltpu.SemaphoreType.DMA,
            pltpu.SemaphoreType.DMA,
            pltpu.SemaphoreType.DMA,
        ],
    )
    def kern(h4_hbm, w_hbm, src_hbm, dst_hbm, agg_hbm,
             srcb0, srcb1, dstb0, dstb1, dst2a, dst2b, offb0, offb1,
             wfl0, wfl1, hr0, hr1, msg, zb, acc_sh, semA, semB, semC, semD):
        c = lax.axis_index("c")
        s = lax.axis_index("s")
        iota = lax.iota(I32, LN)
        zv = iota.astype(F32) * 0.0
        srcb = [srcb0, srcb1]
        dstb = [dstb0, dstb1]
        dst2 = [dst2a, dst2b]
        offb = [offb0, offb1]
        wfl = [wfl0, wfl1]
        hr = [hr0, hr1]
        sem = [semA, semB]
        sem2 = [semC, semD]

        def zrow(i, car):
            zb[i // (CW // LN), pl.ds((i % (CW // LN)) * LN, LN)] = zv
            return car
        lax.fori_loop(0, zr * CW // LN, zrow, 0)

        for k in range(NCHUNK // NC):
            cc = c * (NCHUNK // NC) + k
            ccn = cc * n
            h2 = 2 * cc
            for q in range(rpt // zr):
                pltpu.sync_copy(zb, acc_sh.at[pl.ds(s * rpt + q * zr, zr)])

            @pl.when(s == NS - 1)
            def _():
                pltpu.sync_copy(zb.at[pl.ds(0, tail)],
                                acc_sh.at[pl.ds(rpt * NS, tail)])
            plsc.subcore_barrier()

            def issue(b_i, bsel):
                ebase = s * eb + b_i * bb
                pltpu.sync_copy(src_hbm.at[pl.ds(ebase, bb)], srcb[bsel])

                @plsc.parallel_loop(0, bb // LN, unroll=2)
                def _(i):
                    offb[bsel][pl.ds(i * LN, LN)] = \
                        srcb[bsel][pl.ds(i * LN, LN)] + ccn
                pltpu.async_copy(h4_hbm.at[offb[bsel]], hr[bsel], sem[bsel])
                pltpu.async_copy(dst_hbm.at[pl.ds(ebase, bb)], dstb[bsel],
                                 sem2[bsel])
                pltpu.async_copy(w_hbm.at[pl.ds(ebase * H, bb * H)],
                                 wfl[bsel], sem2[bsel])

            def process(b_i, bsel):
                ebase = s * eb + b_i * bb
                pltpu.make_async_copy(
                    dst_hbm.at[pl.ds(ebase, bb)], dstb[bsel],
                    sem2[bsel]).wait()
                pltpu.make_async_copy(
                    w_hbm.at[pl.ds(ebase * H, bb * H)], wfl[bsel],
                    sem2[bsel]).wait()
                for i in range(bb // LN):
                    dst2[bsel][0, pl.ds(i * LN, LN)] = \
                        dstb[bsel][pl.ds(i * LN, LN)]
                pltpu.make_async_copy(
                    h4_hbm.at[offb[bsel]], hr[bsel], sem[bsel]).wait()

                @plsc.parallel_loop(0, bb // 2, unroll=8)
                def _(p):
                    wv = wfl[bsel][pl.ds(p * LN, LN)]
                    e0 = 2 * p
                    for t in range(2):
                        w0 = _vperm(wv, jnp.full((LN,), 8 * t + h2, I32))
                        w1 = _vperm(wv, jnp.full((LN,), 8 * t + h2 + 1, I32))
                        for j in range(CW // LN):
                            wvv = w0 if j < (CW // LN // 2) else w1
                            hv = hr[bsel][e0 + t, pl.ds(j * LN, LN)]
                            msg[e0 + t, pl.ds(j * LN, LN)] = hv * wvv
                pltpu.sync_copy(msg, acc_sh.at[dst2[bsel].at[0]],
                                add=True)

            issue(0, 0)

            def g_body(g, car):
                b0 = 2 * g
                issue(b0 + 1, 1)
                process(b0, 0)

                @pl.when(b0 + 2 < nblk)
                def _():
                    issue(b0 + 2, 0)
                process(b0 + 1, 1)
                return car
            lax.fori_loop(0, nblk // 2, g_body, 0)

            plsc.subcore_barrier()
            pltpu.sync_copy(acc_sh.at[pl.ds(s * rpt, rpt)],
                            agg_hbm.at[pl.ds(cc * n + s * rpt, rpt)])

            @pl.when(s == NS - 1)
            def _():
                pltpu.sync_copy(acc_sh.at[pl.ds(rpt * NS, tail)],
                                agg_hbm.at[pl.ds(cc * n + rpt * NS, tail)])
            plsc.subcore_barrier()

    return kern


def _attn_mat(a_src, a_dst):
    eye = jnp.eye(H, dtype=F32)
    ms = (a_src[:, :, None] * eye[:, None, :]).reshape(HC, H)
    md = (a_dst[:, :, None] * eye[:, None, :]).reshape(HC, H)
    return jnp.pad(jnp.concatenate([ms, md], axis=1), ((0, 0), (0, 112)))


def _to_chunk_major(h, n):
    return h.reshape(n, NCHUNK, CW).transpose(1, 0, 2).reshape(NCHUNK * n, CW)


def _from_chunk_major(a, n):
    return a.reshape(NCHUNK, n, CW).transpose(1, 0, 2).reshape(n, HC)


def kernel(x, edge_index, edge_attr, batch, W1, a_src1, a_dst1, b1,
           W2, a_src2, a_dst2, b2, W3, a_src3, a_dst3, b3, linW, linb):
    n, _ = x.shape
    e = edge_index.shape[1]
    g = 16
    src = edge_index[0].astype(I32)
    dst = edge_index[1].astype(I32)

    sc_w = _make_sc_edge_w(n, e)
    sc_agg = _make_sc_agg(n, e)
    e8 = jnp.repeat(jnp.eye(H, dtype=F32), C, axis=1)

    def layer(xin, W, a_s, a_d, b):
        h, al = _mm_attn(xin, W, _attn_mat(a_s, a_d))
        we, denp = sc_w(al, src, dst)
        aggf = sc_agg(_to_chunk_major(h, n), we, src, dst)
        agg = _from_chunk_major(aggf, n)
        dp = denp.reshape(2, n, H)
        return _norm_elu(agg, dp[0], dp[1], e8, b.reshape(1, HC))

    x2 = layer(x, W1, a_src1, a_dst1, b1)
    x3 = layer(x2, W2, a_src2, a_dst2, b2)
    x4 = layer(x3, W3, a_src3, a_dst3, b3)

    batch3 = batch.astype(I32).reshape(n // 400, 1, 400)
    ps, cnt = _pool(x4, batch3, g)
    wp = jnp.pad(linW, ((0, 0), (0, 128 - linW.shape[1])))
    bp = jnp.pad(linb, (0, 128 - linb.shape[0])).reshape(1, 128)
    out = _head(ps, cnt, wp, bp, g)
    return out[:, :linW.shape[1]]


# speedup vs baseline: 1.6841x; 1.1915x over previous
import functools

import jax
import jax.numpy as jnp
from jax import lax
from jax.experimental import pallas as pl
from jax.experimental.pallas import tpu as pltpu
from jax.experimental.pallas import tpu_sc as plsc

F32 = jnp.float32
I32 = jnp.int32

NC = 2
NS = 16
LN = 16
NW = NC * NS

H = 8
C = 64
HC = H * C
NCHUNK = 4
CW = HC // NCHUNK


def _mm_attn_body(x_ref, w_ref, am_ref, h_ref, al_ref):
    h = jnp.dot(x_ref[...], w_ref[...], preferred_element_type=F32)
    h_ref[...] = h
    al_ref[...] = jnp.dot(h, am_ref[...], preferred_element_type=F32)


def _mm_attn(x, W, AM, bn=400):
    n, din = x.shape
    hc = W.shape[1]
    return pl.pallas_call(
        _mm_attn_body,
        grid=(n // bn,),
        in_specs=[
            pl.BlockSpec((bn, din), lambda i: (i, 0)),
            pl.BlockSpec((din, hc), lambda i: (0, 0)),
            pl.BlockSpec((hc, 128), lambda i: (0, 0)),
        ],
        out_specs=[
            pl.BlockSpec((bn, hc), lambda i: (i, 0)),
            pl.BlockSpec((bn, 128), lambda i: (i, 0)),
        ],
        out_shape=[
            jax.ShapeDtypeStruct((n, hc), F32),
            jax.ShapeDtypeStruct((n, 128), F32),
        ],
    )(x, W, AM)


def _norm_elu_body(agg_ref, d0_ref, d1_ref, e8_ref, b_ref, out_ref):
    den = d0_ref[...] + d1_ref[...]
    dexp = jnp.dot(den, e8_ref[...], preferred_element_type=F32)
    z = agg_ref[...] / (dexp + 1e-16) + b_ref[...]
    out_ref[...] = jnp.where(z > 0, z, jnp.exp(jnp.minimum(z, 0.0)) - 1.0)


def _norm_elu(agg, d0, d1, e8, b2d, bn=400):
    n = agg.shape[0]
    return pl.pallas_call(
        _norm_elu_body,
        grid=(n // bn,),
        in_specs=[
            pl.BlockSpec((bn, HC), lambda i: (i, 0)),
            pl.BlockSpec((bn, H), lambda i: (i, 0)),
            pl.BlockSpec((bn, H), lambda i: (i, 0)),
            pl.BlockSpec((H, HC), lambda i: (0, 0)),
            pl.BlockSpec((1, HC), lambda i: (0, 0)),
        ],
        out_specs=pl.BlockSpec((bn, HC), lambda i: (i, 0)),
        out_shape=jax.ShapeDtypeStruct((n, HC), F32),
    )(agg, d0, d1, e8, b2d)


def _pool_body(x_ref, b_ref, ps_ref, cnt_ref):
    i = pl.program_id(0)

    @pl.when(i == 0)
    def _():
        ps_ref[...] = jnp.zeros_like(ps_ref)
        cnt_ref[...] = jnp.zeros_like(cnt_ref)

    bn = x_ref.shape[0]
    g = ps_ref.shape[0]
    bb = jnp.broadcast_to(b_ref[...].reshape(1, bn), (g, bn))
    gi = lax.broadcasted_iota(I32, (g, bn), 0)
    p = (bb == gi).astype(F32)
    ps_ref[...] += jnp.dot(p, x_ref[...], preferred_element_type=F32)
    cnt_ref[...] += jnp.broadcast_to(
        jnp.sum(p, axis=1, keepdims=True), cnt_ref.shape)


def _pool(x, batch3, g, bn=400):
    n = x.shape[0]
    return pl.pallas_call(
        _pool_body,
        grid=(n // bn,),
        in_specs=[
            pl.BlockSpec((bn, HC), lambda i: (i, 0)),
            pl.BlockSpec((1, 1, bn), lambda i: (i, 0, 0)),
        ],
        out_specs=[
            pl.BlockSpec((g, HC), lambda i: (0, 0)),
            pl.BlockSpec((g, 128), lambda i: (0, 0)),
        ],
        out_shape=[
            jax.ShapeDtypeStruct((g, HC), F32),
            jax.ShapeDtypeStruct((g, 128), F32),
        ],
    )(x, batch3)


def _head_body(ps_ref, cnt_ref, w_ref, b_ref, out_ref):
    cnt = jnp.maximum(cnt_ref[:, 0:1], 1.0)
    pooled = ps_ref[...] / cnt
    logits = jnp.dot(pooled, w_ref[...], preferred_element_type=F32) + b_ref[...]
    mask = lax.broadcasted_iota(I32, logits.shape, 1) < 10
    logits = jnp.where(mask, logits, -1e30)
    m = jnp.max(logits, axis=1, keepdims=True)
    lse = m + jnp.log(jnp.sum(jnp.exp(logits - m), axis=1, keepdims=True))
    out_ref[...] = logits - lse


def _head(ps, cnt, wp, bp, g):
    return pl.pallas_call(
        _head_body,
        grid=(1,),
        in_specs=[
            pl.BlockSpec((g, HC), lambda i: (0, 0)),
            pl.BlockSpec((g, 128), lambda i: (0, 0)),
            pl.BlockSpec((HC, 128), lambda i: (0, 0)),
            pl.BlockSpec((1, 128), lambda i: (0, 0)),
        ],
        out_specs=pl.BlockSpec((g, 128), lambda i: (0, 0)),
        out_shape=jax.ShapeDtypeStruct((g, 128), F32),
    )(ps, cnt, wp, bp)


_GDN = lax.GatherDimensionNumbers(
    offset_dims=(), collapsed_slice_dims=(0,), start_index_map=(0,))


def _vperm(v, idx):
    return lax.gather(v, idx[:, None], _GDN, (1,),
                      mode=lax.GatherScatterMode.PROMISE_IN_BOUNDS)

def _make_sc_edge_w(n, e):
    eb = e // NW
    ba = 80
    nblk = eb // ba
    nfl = n * H
    st = nfl // NS
    mesh = plsc.VectorSubcoreMesh(core_axis_name="c", subcore_axis_name="s")

    @functools.partial(
        pl.kernel,
        out_type=[
            jax.ShapeDtypeStruct((e * H,), F32),
            jax.ShapeDtypeStruct((2 * nfl,), F32),
        ],
        mesh=mesh,
        scratch_types=[
            pltpu.VMEM((ba,), I32),
            pltpu.VMEM((ba,), I32),
            pltpu.VMEM((ba,), I32),
            pltpu.VMEM((ba,), I32),
            pltpu.VMEM((ba, 128), F32),
            pltpu.VMEM((ba, 128), F32),
            pltpu.VMEM((ba, 128), F32),
            pltpu.VMEM((ba, 128), F32),
            pltpu.VMEM((ba * H,), F32),
            pltpu.VMEM((ba * H,), F32),
            pltpu.VMEM((ba * H // 128, 128), I32),
            pltpu.VMEM((ba * H // 128, 128), I32),
            pltpu.VMEM((st + 16, ), F32),
            pltpu.VMEM_SHARED((nfl,), F32),
            pltpu.SemaphoreType.DMA,
            pltpu.SemaphoreType.DMA,
            pltpu.SemaphoreType.DMA,
            pltpu.SemaphoreType.DMA,
        ],
    )
    def kern(al_hbm, src_hbm, dst_hbm, w_hbm, denp_hbm,
             srcb0, srcb1, dstb0, dstb1, asb0, asb1, adb0, adb1,
             wfl0, wfl1, idx0, idx1, zb, den_sh, semA, semB, semC, semD):
        c = lax.axis_index("c")
        s = lax.axis_index("s")
        wid = c * NS + s
        iota = lax.iota(I32, LN)
        zv = iota.astype(F32) * 0.0
        i8 = iota & 7
        hi = iota >> 3
        lo8 = iota < 8
        srcb = [srcb0, srcb1]
        dstb = [dstb0, dstb1]
        asb = [asb0, asb1]
        adb = [adb0, adb1]
        wfl = [wfl0, wfl1]
        idx2 = [idx0, idx1]
        semg = [semA, semB]
        sems = [semC, semD]

        def zrow(i, car):
            zb[pl.ds(i * LN, LN)] = zv
            return car
        lax.fori_loop(0, (st + 16) // LN, zrow, 0)
        pltpu.sync_copy(zb.at[pl.ds(0, st)], den_sh.at[pl.ds(s * st, st)])
        plsc.subcore_barrier()

        def issue(b_i, bsel):
            ebase = wid * eb + b_i * ba
            pltpu.async_copy(src_hbm.at[pl.ds(ebase, ba)], srcb[bsel],
                             sems[bsel])
            pltpu.async_copy(dst_hbm.at[pl.ds(ebase, ba)], dstb[bsel],
                             sems[bsel])

        def gstage(b_i, bsel):
            ebase = wid * eb + b_i * ba
            pltpu.make_async_copy(src_hbm.at[pl.ds(ebase, ba)], srcb[bsel],
                                  sems[bsel]).wait()
            pltpu.make_async_copy(dst_hbm.at[pl.ds(ebase, ba)], dstb[bsel],
                                  sems[bsel]).wait()
            pltpu.async_copy(al_hbm.at[srcb[bsel]], asb[bsel], semg[bsel])
            pltpu.async_copy(al_hbm.at[dstb[bsel]], adb[bsel], semg[bsel])

        def process(b_i, bsel):
            ebase = wid * eb + b_i * ba
            pltpu.make_async_copy(al_hbm.at[srcb[bsel]], asb[bsel],
                                  semg[bsel]).wait()
            pltpu.make_async_copy(al_hbm.at[dstb[bsel]], adb[bsel],
                                  semg[bsel]).wait()

            @plsc.parallel_loop(0, ba // 2, unroll=4)
            def _(p):
                e0 = 2 * p
                a0 = _vperm(asb[bsel][e0, pl.ds(0, LN)], i8)
                a1 = _vperm(asb[bsel][e0 + 1, pl.ds(0, LN)], i8)
                ga = jnp.where(lo8, a0, a1)
                d0 = _vperm(adb[bsel][e0, pl.ds(0, LN)], i8 + 8)
                d1 = _vperm(adb[bsel][e0 + 1, pl.ds(0, LN)], i8 + 8)
                gd = jnp.where(lo8, d0, d1)
                sv = ga + gd
                lv = jnp.maximum(sv, sv * 0.2)
                wfl[bsel][pl.ds(p * LN, LN)] = jnp.exp(lv)

            @plsc.parallel_loop(0, ba // LN, unroll=2)
            def _(i):
                dv = dstb[bsel][pl.ds(i * LN, LN)]
                for j in range(8):
                    idxv = _vperm(dv, 2 * j + hi) * H + i8
                    idx2[bsel][i, pl.ds(j * LN, LN)] = idxv

            pltpu.sync_copy(wfl[bsel], w_hbm.at[pl.ds(ebase * H, ba * H)])
            for q in range(ba * H // 128):
                pltpu.sync_copy(wfl[bsel].at[pl.ds(q * 128, 128)],
                                den_sh.at[idx2[bsel].at[q]], add=True)

        issue(0, 0)
        gstage(0, 0)
        issue(1, 1)

        def g_body(g, car):
            b0 = 2 * g
            gstage(b0 + 1, 1)
            process(b0, 0)

            @pl.when(b0 + 2 < nblk)
            def _():
                issue(b0 + 2, 0)
                gstage(b0 + 2, 0)
            process(b0 + 1, 1)

            @pl.when(b0 + 3 < nblk)
            def _():
                issue(b0 + 3, 1)
            return car
        lax.fori_loop(0, nblk // 2, g_body, 0)
        if nblk % 2 == 1:
            process(nblk - 1, 0)

        plsc.subcore_barrier()
        pltpu.sync_copy(den_sh.at[pl.ds(s * st, st)], zb.at[pl.ds(0, st)])
        pltpu.sync_copy(zb.at[pl.ds(0, st)],
                        denp_hbm.at[pl.ds(c * nfl + s * st, st)])

    return kern


def _make_sc_agg(n, e):
    eb = e // NS
    bb = 80
    nblk = eb // bb
    rpt = 624
    tail = n - rpt * NS
    zr = 48
    mesh = plsc.VectorSubcoreMesh(core_axis_name="c", subcore_axis_name="s")

    @functools.partial(
        pl.kernel,
        out_type=jax.ShapeDtypeStruct((NCHUNK * n, CW), F32),
        mesh=mesh,
        scratch_types=[
            pltpu.VMEM((bb,), I32),
            pltpu.VMEM((bb,), I32),
            pltpu.VMEM((bb,), I32),
            pltpu.VMEM((bb,), I32),
            pltpu.VMEM((1, bb), I32),
            pltpu.VMEM((1, bb), I32),
            pltpu.VMEM((bb,), I32),
            pltpu.VMEM((bb,), I32),
            pltpu.VMEM((bb * H,), F32),
            pltpu.VMEM((bb * H,), F32),
            pltpu.VMEM((bb, CW), F32),
            pltpu.VMEM((bb, CW), F32),
            pltpu.VMEM((bb, CW), F32),
            pltpu.VMEM((zr, CW), F32),
            pltpu.VMEM_SHARED((n, CW), F32),
            pltpu.SemaphoreType.DMA,
            pltpu.SemaphoreType.DMA,
            pltpu.SemaphoreType.DMA,
            pltpu.SemaphoreType.DMA,
        ],
    )
    def kern(h4_hbm, w_hbm, src_hbm, dst_hbm, agg_hbm,
             srcb0, srcb1, dstb0, dstb1, dst2a, dst2b, offb0, offb1,
             wfl0, wfl1, hr0, hr1, msg, zb, acc_sh, semA, semB, semC, semD):
        c = lax.axis_index("c")
        s = lax.axis_index("s")
        iota = lax.iota(I32, LN)
        zv = iota.astype(F32) * 0.0
        srcb = [srcb0, srcb1]
        dstb = [dstb0, dstb1]
        dst2 = [dst2a, dst2b]
        offb = [offb0, offb1]
        wfl = [wfl0, wfl1]
        hr = [hr0, hr1]
        sem = [semA, semB]
        sem2 = [semC, semD]

        def zrow(i, car):
            zb[i // (CW // LN), pl.ds((i % (CW // LN)) * LN, LN)] = zv
            return car
        lax.fori_loop(0, zr * CW // LN, zrow, 0)

        for k in range(NCHUNK // NC):
            cc = c * (NCHUNK // NC) + k
            ccn = cc * n
            h2 = 2 * cc
            for q in range(rpt // zr):
                pltpu.sync_copy(zb, acc_sh.at[pl.ds(s * rpt + q * zr, zr)])

            @pl.when(s == NS - 1)
            def _():
                pltpu.sync_copy(zb.at[pl.ds(0, tail)],
                                acc_sh.at[pl.ds(rpt * NS, tail)])
            plsc.subcore_barrier()

            def issue(b_i, bsel):
                ebase = s * eb + b_i * bb
                pltpu.sync_copy(src_hbm.at[pl.ds(ebase, bb)], srcb[bsel])

                @plsc.parallel_loop(0, bb // LN, unroll=2)
                def _(i):
                    offb[bsel][pl.ds(i * LN, LN)] = \
                        srcb[bsel][pl.ds(i * LN, LN)] + ccn
                pltpu.async_copy(h4_hbm.at[offb[bsel]], hr[bsel], sem[bsel])
                pltpu.async_copy(dst_hbm.at[pl.ds(ebase, bb)], dstb[bsel],
                                 sem2[bsel])
                pltpu.async_copy(w_hbm.at[pl.ds(ebase * H, bb * H)],
                                 wfl[bsel], sem2[bsel])

            def process(b_i, bsel):
                ebase = s * eb + b_i * bb
                pltpu.make_async_copy(
                    dst_hbm.at[pl.ds(ebase, bb)], dstb[bsel],
                    sem2[bsel]).wait()
                pltpu.make_async_copy(
                    w_hbm.at[pl.ds(ebase * H, bb * H)], wfl[bsel],
                    sem2[bsel]).wait()
                for i in range(bb // LN):
                    dst2[bsel][0, pl.ds(i * LN, LN)] = \
                        dstb[bsel][pl.ds(i * LN, LN)]
                pltpu.make_async_copy(
                    h4_hbm.at[offb[bsel]], hr[bsel], sem[bsel]).wait()

                @plsc.parallel_loop(0, bb // 2, unroll=8)
                def _(p):
                    wv = wfl[bsel][pl.ds(p * LN, LN)]
                    e0 = 2 * p
                    for t in range(2):
                        w0 = _vperm(wv, jnp.full((LN,), 8 * t + h2, I32))
                        w1 = _vperm(wv, jnp.full((LN,), 8 * t + h2 + 1, I32))
                        for j in range(CW // LN):
                            wvv = w0 if j < (CW // LN // 2) else w1
                            hv = hr[bsel][e0 + t, pl.ds(j * LN, LN)]
                            msg[e0 + t, pl.ds(j * LN, LN)] = hv * wvv
                pltpu.sync_copy(msg, acc_sh.at[dst2[bsel].at[0]],
                                add=True)

            issue(0, 0)

            def g_body(g, car):
                b0 = 2 * g
                issue(b0 + 1, 1)
                process(b0, 0)

                @pl.when(b0 + 2 < nblk)
                def _():
                    issue(b0 + 2, 0)
                process(b0 + 1, 1)
                return car
            lax.fori_loop(0, nblk // 2, g_body, 0)

            plsc.subcore_barrier()
            pltpu.sync_copy(acc_sh.at[pl.ds(s * rpt, rpt)],
                            agg_hbm.at[pl.ds(cc * n + s * rpt, rpt)])

            @pl.when(s == NS - 1)
            def _():
                pltpu.sync_copy(acc_sh.at[pl.ds(rpt * NS, tail)],
                                agg_hbm.at[pl.ds(cc * n + rpt * NS, tail)])
            plsc.subcore_barrier()

    return kern


def _attn_mat(a_src, a_dst):
    eye = jnp.eye(H, dtype=F32)
    ms = (a_src[:, :, None] * eye[:, None, :]).reshape(HC, H)
    md = (a_dst[:, :, None] * eye[:, None, :]).reshape(HC, H)
    return jnp.pad(jnp.concatenate([ms, md], axis=1), ((0, 0), (0, 112)))


def _to_chunk_major(h, n):
    return h.reshape(n, NCHUNK, CW).transpose(1, 0, 2).reshape(NCHUNK * n, CW)


def _from_chunk_major(a, n):
    return a.reshape(NCHUNK, n, CW).transpose(1, 0, 2).reshape(n, HC)


def kernel(x, edge_index, edge_attr, batch, W1, a_src1, a_dst1, b1,
           W2, a_src2, a_dst2, b2, W3, a_src3, a_dst3, b3, linW, linb):
    n, _ = x.shape
    e = edge_index.shape[1]
    g = 16
    src = edge_index[0].astype(I32)
    dst = edge_index[1].astype(I32)

    sc_w = _make_sc_edge_w(n, e)
    sc_agg = _make_sc_agg(n, e)
    e8 = jnp.repeat(jnp.eye(H, dtype=F32), C, axis=1)

    def layer(xin, W, a_s, a_d, b):
        h, al = _mm_attn(xin, W, _attn_mat(a_s, a_d))
        we, denp = sc_w(al, src, dst)
        aggf = sc_agg(_to_chunk_major(h, n), we, src, dst)
        agg = _from_chunk_major(aggf, n)
        dp = denp.reshape(2, n, H)
        return _norm_elu(agg, dp[0], dp[1], e8, b.reshape(1, HC))

    x2 = layer(x, W1, a_src1, a_dst1, b1)
    x3 = layer(x2, W2, a_src2, a_dst2, b2)
    x4 = layer(x3, W3, a_src3, a_dst3, b3)

    batch3 = batch.astype(I32).reshape(n // 400, 1, 400)
    ps, cnt = _pool(x4, batch3, g)
    wp = jnp.pad(linW, ((0, 0), (0, 128 - linW.shape[1])))
    bp = jnp.pad(linb, (0, 128 - linb.shape[0])).reshape(1, 128)
    out = _head(ps, cnt, wp, bp, g)
    return out[:, :linW.shape[1]]


# speedup vs baseline: 1.9530x; 1.1597x over previous
import functools

import jax
import jax.numpy as jnp
from jax import lax
from jax.experimental import pallas as pl
from jax.experimental.pallas import tpu as pltpu
from jax.experimental.pallas import tpu_sc as plsc

F32 = jnp.float32
I32 = jnp.int32

NC = 2
NS = 16
LN = 16
NW = NC * NS

H = 8
C = 64
HC = H * C
NCHUNK = 4
CW = HC // NCHUNK


def _mm_attn_body(x_ref, w_ref, am_ref, h_ref, al_ref):
    h = jnp.dot(x_ref[...], w_ref[...], preferred_element_type=F32)
    h_ref[...] = h
    al_ref[...] = jnp.dot(h, am_ref[...], preferred_element_type=F32)


def _mm_attn(x, W, AM, bn=400):
    n, din = x.shape
    hc = W.shape[1]
    return pl.pallas_call(
        _mm_attn_body,
        grid=(n // bn,),
        in_specs=[
            pl.BlockSpec((bn, din), lambda i: (i, 0)),
            pl.BlockSpec((din, hc), lambda i: (0, 0)),
            pl.BlockSpec((hc, 128), lambda i: (0, 0)),
        ],
        out_specs=[
            pl.BlockSpec((bn, hc), lambda i: (i, 0)),
            pl.BlockSpec((bn, 128), lambda i: (i, 0)),
        ],
        out_shape=[
            jax.ShapeDtypeStruct((n, hc), F32),
            jax.ShapeDtypeStruct((n, 128), F32),
        ],
    )(x, W, AM)


def _norm_elu_body(agg_ref, d0_ref, d1_ref, e8_ref, b_ref, out_ref):
    den = d0_ref[...] + d1_ref[...]
    dexp = jnp.dot(den, e8_ref[...], preferred_element_type=F32)
    z = agg_ref[...] / (dexp + 1e-16) + b_ref[...]
    out_ref[...] = jnp.where(z > 0, z, jnp.exp(jnp.minimum(z, 0.0)) - 1.0)


def _norm_elu(agg, d0, d1, e8, b2d, bn=400):
    n = agg.shape[0]
    return pl.pallas_call(
        _norm_elu_body,
        grid=(n // bn,),
        in_specs=[
            pl.BlockSpec((bn, HC), lambda i: (i, 0)),
            pl.BlockSpec((bn, H), lambda i: (i, 0)),
            pl.BlockSpec((bn, H), lambda i: (i, 0)),
            pl.BlockSpec((H, HC), lambda i: (0, 0)),
            pl.BlockSpec((1, HC), lambda i: (0, 0)),
        ],
        out_specs=pl.BlockSpec((bn, HC), lambda i: (i, 0)),
        out_shape=jax.ShapeDtypeStruct((n, HC), F32),
    )(agg, d0, d1, e8, b2d)


def _pool_body(x_ref, b_ref, ps_ref, cnt_ref):
    i = pl.program_id(0)

    @pl.when(i == 0)
    def _():
        ps_ref[...] = jnp.zeros_like(ps_ref)
        cnt_ref[...] = jnp.zeros_like(cnt_ref)

    bn = x_ref.shape[0]
    g = ps_ref.shape[0]
    bb = jnp.broadcast_to(b_ref[...].reshape(1, bn), (g, bn))
    gi = lax.broadcasted_iota(I32, (g, bn), 0)
    p = (bb == gi).astype(F32)
    ps_ref[...] += jnp.dot(p, x_ref[...], preferred_element_type=F32)
    cnt_ref[...] += jnp.broadcast_to(
        jnp.sum(p, axis=1, keepdims=True), cnt_ref.shape)


def _pool(x, batch3, g, bn=400):
    n = x.shape[0]
    return pl.pallas_call(
        _pool_body,
        grid=(n // bn,),
        in_specs=[
            pl.BlockSpec((bn, HC), lambda i: (i, 0)),
            pl.BlockSpec((1, 1, bn), lambda i: (i, 0, 0)),
        ],
        out_specs=[
            pl.BlockSpec((g, HC), lambda i: (0, 0)),
            pl.BlockSpec((g, 128), lambda i: (0, 0)),
        ],
        out_shape=[
            jax.ShapeDtypeStruct((g, HC), F32),
            jax.ShapeDtypeStruct((g, 128), F32),
        ],
    )(x, batch3)


def _head_body(ps_ref, cnt_ref, w_ref, b_ref, out_ref):
    cnt = jnp.maximum(cnt_ref[:, 0:1], 1.0)
    pooled = ps_ref[...] / cnt
    logits = jnp.dot(pooled, w_ref[...], preferred_element_type=F32) + b_ref[...]
    mask = lax.broadcasted_iota(I32, logits.shape, 1) < 10
    logits = jnp.where(mask, logits, -1e30)
    m = jnp.max(logits, axis=1, keepdims=True)
    lse = m + jnp.log(jnp.sum(jnp.exp(logits - m), axis=1, keepdims=True))
    out_ref[...] = logits - lse


def _head(ps, cnt, wp, bp, g):
    return pl.pallas_call(
        _head_body,
        grid=(1,),
        in_specs=[
            pl.BlockSpec((g, HC), lambda i: (0, 0)),
            pl.BlockSpec((g, 128), lambda i: (0, 0)),
            pl.BlockSpec((HC, 128), lambda i: (0, 0)),
            pl.BlockSpec((1, 128), lambda i: (0, 0)),
        ],
        out_specs=pl.BlockSpec((g, 128), lambda i: (0, 0)),
        out_shape=jax.ShapeDtypeStruct((g, 128), F32),
    )(ps, cnt, wp, bp)


_GDN = lax.GatherDimensionNumbers(
    offset_dims=(), collapsed_slice_dims=(0,), start_index_map=(0,))


def _vperm(v, idx):
    return lax.gather(v, idx[:, None], _GDN, (1,),
                      mode=lax.GatherScatterMode.PROMISE_IN_BOUNDS)

def _make_sc_edge_w(n, e):
    eb = e // NW
    ba = 80
    nblk = eb // ba
    nfl = n * H
    st = nfl // NS
    mesh = plsc.VectorSubcoreMesh(core_axis_name="c", subcore_axis_name="s")

    @functools.partial(
        pl.kernel,
        out_type=[
            jax.ShapeDtypeStruct((e * H,), F32),
            jax.ShapeDtypeStruct((2 * nfl,), F32),
        ],
        mesh=mesh,
        scratch_types=[
            pltpu.VMEM((ba,), I32),
            pltpu.VMEM((ba,), I32),
            pltpu.VMEM((ba,), I32),
            pltpu.VMEM((ba,), I32),
            pltpu.VMEM((ba, 128), F32),
            pltpu.VMEM((ba, 128), F32),
            pltpu.VMEM((ba, 128), F32),
            pltpu.VMEM((ba, 128), F32),
            pltpu.VMEM((ba * H,), F32),
            pltpu.VMEM((ba * H,), F32),
            pltpu.VMEM((ba * H // 128, 128), I32),
            pltpu.VMEM((ba * H // 128, 128), I32),
            pltpu.VMEM((st + 16, ), F32),
            pltpu.VMEM_SHARED((nfl,), F32),
            pltpu.SemaphoreType.DMA,
            pltpu.SemaphoreType.DMA,
            pltpu.SemaphoreType.DMA,
            pltpu.SemaphoreType.DMA,
        ],
    )
    def kern(al_hbm, src_hbm, dst_hbm, w_hbm, denp_hbm,
             srcb0, srcb1, dstb0, dstb1, asb0, asb1, adb0, adb1,
             wfl0, wfl1, idx0, idx1, zb, den_sh, semA, semB, semC, semD):
        c = lax.axis_index("c")
        s = lax.axis_index("s")
        wid = c * NS + s
        iota = lax.iota(I32, LN)
        zv = iota.astype(F32) * 0.0
        i8 = iota & 7
        hi = iota >> 3
        lo8 = iota < 8
        srcb = [srcb0, srcb1]
        dstb = [dstb0, dstb1]
        asb = [asb0, asb1]
        adb = [adb0, adb1]
        wfl = [wfl0, wfl1]
        idx2 = [idx0, idx1]
        semg = [semA, semB]
        sems = [semC, semD]

        def zrow(i, car):
            zb[pl.ds(i * LN, LN)] = zv
            return car
        lax.fori_loop(0, (st + 16) // LN, zrow, 0)
        pltpu.sync_copy(zb.at[pl.ds(0, st)], den_sh.at[pl.ds(s * st, st)])
        plsc.subcore_barrier()

        def issue(b_i, bsel):
            ebase = wid * eb + b_i * ba
            pltpu.async_copy(src_hbm.at[pl.ds(ebase, ba)], srcb[bsel],
                             sems[bsel])
            pltpu.async_copy(dst_hbm.at[pl.ds(ebase, ba)], dstb[bsel],
                             sems[bsel])

        def gstage(b_i, bsel):
            ebase = wid * eb + b_i * ba
            pltpu.make_async_copy(src_hbm.at[pl.ds(ebase, ba)], srcb[bsel],
                                  sems[bsel]).wait()
            pltpu.make_async_copy(dst_hbm.at[pl.ds(ebase, ba)], dstb[bsel],
                                  sems[bsel]).wait()
            pltpu.async_copy(al_hbm.at[srcb[bsel]], asb[bsel], semg[bsel])
            pltpu.async_copy(al_hbm.at[dstb[bsel]], adb[bsel], semg[bsel])

        def process(b_i, bsel):
            ebase = wid * eb + b_i * ba
            pltpu.make_async_copy(al_hbm.at[srcb[bsel]], asb[bsel],
                                  semg[bsel]).wait()
            pltpu.make_async_copy(al_hbm.at[dstb[bsel]], adb[bsel],
                                  semg[bsel]).wait()

            @plsc.parallel_loop(0, ba // 2, unroll=4)
            def _(p):
                e0 = 2 * p
                a0 = _vperm(asb[bsel][e0, pl.ds(0, LN)], i8)
                a1 = _vperm(asb[bsel][e0 + 1, pl.ds(0, LN)], i8)
                ga = jnp.where(lo8, a0, a1)
                d0 = _vperm(adb[bsel][e0, pl.ds(0, LN)], i8 + 8)
                d1 = _vperm(adb[bsel][e0 + 1, pl.ds(0, LN)], i8 + 8)
                gd = jnp.where(lo8, d0, d1)
                sv = ga + gd
                lv = jnp.maximum(sv, sv * 0.2)
                wfl[bsel][pl.ds(p * LN, LN)] = jnp.exp(lv)

            @plsc.parallel_loop(0, ba // LN, unroll=2)
            def _(i):
                dv = dstb[bsel][pl.ds(i * LN, LN)]
                for j in range(8):
                    idxv = _vperm(dv, 2 * j + hi) * H + i8
                    idx2[bsel][i, pl.ds(j * LN, LN)] = idxv

            pltpu.sync_copy(wfl[bsel], w_hbm.at[pl.ds(ebase * H, ba * H)])
            for q in range(ba * H // 128):
                pltpu.sync_copy(wfl[bsel].at[pl.ds(q * 128, 128)],
                                den_sh.at[idx2[bsel].at[q]], add=True)

        issue(0, 0)
        gstage(0, 0)
        issue(1, 1)

        def g_body(g, car):
            b0 = 2 * g
            gstage(b0 + 1, 1)
            process(b0, 0)

            @pl.when(b0 + 2 < nblk)
            def _():
                issue(b0 + 2, 0)
                gstage(b0 + 2, 0)
            process(b0 + 1, 1)

            @pl.when(b0 + 3 < nblk)
            def _():
                issue(b0 + 3, 1)
            return car
        lax.fori_loop(0, nblk // 2, g_body, 0)
        if nblk % 2 == 1:
            process(nblk - 1, 0)

        plsc.subcore_barrier()
        pltpu.sync_copy(den_sh.at[pl.ds(s * st, st)], zb.at[pl.ds(0, st)])
        pltpu.sync_copy(zb.at[pl.ds(0, st)],
                        denp_hbm.at[pl.ds(c * nfl + s * st, st)])

    return kern


def _make_sc_agg(n, e):
    eb = e // NS
    bb = 80
    nblk = eb // bb
    rpt = 624
    tail = n - rpt * NS
    zr = 16
    mesh = plsc.VectorSubcoreMesh(core_axis_name="c", subcore_axis_name="s")

    @functools.partial(
        pl.kernel,
        out_type=jax.ShapeDtypeStruct((NCHUNK * n, CW), F32),
        mesh=mesh,
        scratch_types=[
            pltpu.VMEM((bb,), I32),
            pltpu.VMEM((bb,), I32),
            pltpu.VMEM((bb,), I32),
            pltpu.VMEM((bb,), I32),
            pltpu.VMEM((1, bb), I32),
            pltpu.VMEM((1, bb), I32),
            pltpu.VMEM((bb,), I32),
            pltpu.VMEM((bb,), I32),
            pltpu.VMEM((bb * H,), F32),
            pltpu.VMEM((bb * H,), F32),
            pltpu.VMEM((bb, CW), F32),
            pltpu.VMEM((bb, CW), F32),
            pltpu.VMEM((bb, CW), F32),
            pltpu.VMEM((bb, CW), F32),
            pltpu.VMEM((zr, CW), F32),
            pltpu.VMEM_SHARED((n, CW), F32),
            pltpu.SemaphoreType.DMA,
            pltpu.SemaphoreType.DMA,
            pltpu.SemaphoreType.DMA,
            pltpu.SemaphoreType.DMA,
            pltpu.SemaphoreType.DMA,
            pltpu.SemaphoreType.DMA,
        ],
    )
    def kern(h4_hbm, w_hbm, src_hbm, dst_hbm, agg_hbm,
             srcb0, srcb1, dstb0, dstb1, dst2a, dst2b, offb0, offb1,
             wfl0, wfl1, hr0, hr1, msg0, msg1, zb, acc_sh,
             semA, semB, semC, semD, semE, semF):
        c = lax.axis_index("c")
        s = lax.axis_index("s")
        iota = lax.iota(I32, LN)
        zv = iota.astype(F32) * 0.0
        srcb = [srcb0, srcb1]
        dstb = [dstb0, dstb1]
        dst2 = [dst2a, dst2b]
        offb = [offb0, offb1]
        wfl = [wfl0, wfl1]
        hr = [hr0, hr1]
        sem = [semA, semB]
        sem2 = [semC, semD]
        msg = [msg0, msg1]
        sem3 = [semE, semF]

        def zrow(i, car):
            zb[i // (CW // LN), pl.ds((i % (CW // LN)) * LN, LN)] = zv
            return car
        lax.fori_loop(0, zr * CW // LN, zrow, 0)

        for k in range(NCHUNK // NC):
            cc = c * (NCHUNK // NC) + k
            ccn = cc * n
            h2 = 2 * cc
            for q in range(rpt // zr):
                pltpu.sync_copy(zb, acc_sh.at[pl.ds(s * rpt + q * zr, zr)])

            @pl.when(s == NS - 1)
            def _():
                pltpu.sync_copy(zb.at[pl.ds(0, tail)],
                                acc_sh.at[pl.ds(rpt * NS, tail)])
            plsc.subcore_barrier()

            def issue(b_i, bsel):
                ebase = s * eb + b_i * bb
                pltpu.sync_copy(src_hbm.at[pl.ds(ebase, bb)], srcb[bsel])

                @plsc.parallel_loop(0, bb // LN, unroll=2)
                def _(i):
                    offb[bsel][pl.ds(i * LN, LN)] = \
                        srcb[bsel][pl.ds(i * LN, LN)] + ccn
                pltpu.async_copy(h4_hbm.at[offb[bsel]], hr[bsel], sem[bsel])
                pltpu.async_copy(dst_hbm.at[pl.ds(ebase, bb)], dstb[bsel],
                                 sem2[bsel])
                pltpu.async_copy(w_hbm.at[pl.ds(ebase * H, bb * H)],
                                 wfl[bsel], sem2[bsel])

            def process(b_i, bsel):
                ebase = s * eb + b_i * bb
                pltpu.make_async_copy(
                    dst_hbm.at[pl.ds(ebase, bb)], dstb[bsel],
                    sem2[bsel]).wait()
                pltpu.make_async_copy(
                    w_hbm.at[pl.ds(ebase * H, bb * H)], wfl[bsel],
                    sem2[bsel]).wait()
                for i in range(bb // LN):
                    dst2[bsel][0, pl.ds(i * LN, LN)] = \
                        dstb[bsel][pl.ds(i * LN, LN)]
                pltpu.make_async_copy(
                    h4_hbm.at[offb[bsel]], hr[bsel], sem[bsel]).wait()

                @pl.when(b_i >= 2)
                def _():
                    pltpu.make_async_copy(
                        msg[bsel], acc_sh.at[dst2[bsel].at[0]],
                        sem3[bsel]).wait()

                @plsc.parallel_loop(0, bb // 2, unroll=8)
                def _(p):
                    wv = wfl[bsel][pl.ds(p * LN, LN)]
                    e0 = 2 * p
                    for t in range(2):
                        w0 = _vperm(wv, jnp.full((LN,), 8 * t + h2, I32))
                        w1 = _vperm(wv, jnp.full((LN,), 8 * t + h2 + 1, I32))
                        for j in range(CW // LN):
                            wvv = w0 if j < (CW // LN // 2) else w1
                            hv = hr[bsel][e0 + t, pl.ds(j * LN, LN)]
                            msg[bsel][e0 + t, pl.ds(j * LN, LN)] = hv * wvv
                pltpu.async_copy(msg[bsel], acc_sh.at[dst2[bsel].at[0]],
                                 sem3[bsel], add=True)

            issue(0, 0)

            def g_body(g, car):
                b0 = 2 * g
                issue(b0 + 1, 1)
                process(b0, 0)

                @pl.when(b0 + 2 < nblk)
                def _():
                    issue(b0 + 2, 0)
                process(b0 + 1, 1)
                return car
            lax.fori_loop(0, nblk // 2, g_body, 0)
            for bsel in range(2):
                pltpu.make_async_copy(
                    msg[bsel], acc_sh.at[dst2[bsel].at[0]],
                    sem3[bsel]).wait()

            plsc.subcore_barrier()
            pltpu.sync_copy(acc_sh.at[pl.ds(s * rpt, rpt)],
                            agg_hbm.at[pl.ds(cc * n + s * rpt, rpt)])

            @pl.when(s == NS - 1)
            def _():
                pltpu.sync_copy(acc_sh.at[pl.ds(rpt * NS, tail)],
                                agg_hbm.at[pl.ds(cc * n + rpt * NS, tail)])
            plsc.subcore_barrier()

    return kern


def _attn_mat(a_src, a_dst):
    eye = jnp.eye(H, dtype=F32)
    ms = (a_src[:, :, None] * eye[:, None, :]).reshape(HC, H)
    md = (a_dst[:, :, None] * eye[:, None, :]).reshape(HC, H)
    return jnp.pad(jnp.concatenate([ms, md], axis=1), ((0, 0), (0, 112)))


def _to_chunk_major(h, n):
    return h.reshape(n, NCHUNK, CW).transpose(1, 0, 2).reshape(NCHUNK * n, CW)


def _from_chunk_major(a, n):
    return a.reshape(NCHUNK, n, CW).transpose(1, 0, 2).reshape(n, HC)


def kernel(x, edge_index, edge_attr, batch, W1, a_src1, a_dst1, b1,
           W2, a_src2, a_dst2, b2, W3, a_src3, a_dst3, b3, linW, linb):
    n, _ = x.shape
    e = edge_index.shape[1]
    g = 16
    src = edge_index[0].astype(I32)
    dst = edge_index[1].astype(I32)

    sc_w = _make_sc_edge_w(n, e)
    sc_agg = _make_sc_agg(n, e)
    e8 = jnp.repeat(jnp.eye(H, dtype=F32), C, axis=1)

    def layer(xin, W, a_s, a_d, b):
        h, al = _mm_attn(xin, W, _attn_mat(a_s, a_d))
        we, denp = sc_w(al, src, dst)
        aggf = sc_agg(_to_chunk_major(h, n), we, src, dst)
        agg = _from_chunk_major(aggf, n)
        dp = denp.reshape(2, n, H)
        return _norm_elu(agg, dp[0], dp[1], e8, b.reshape(1, HC))

    x2 = layer(x, W1, a_src1, a_dst1, b1)
    x3 = layer(x2, W2, a_src2, a_dst2, b2)
    x4 = layer(x3, W3, a_src3, a_dst3, b3)

    batch3 = batch.astype(I32).reshape(n // 400, 1, 400)
    ps, cnt = _pool(x4, batch3, g)
    wp = jnp.pad(linW, ((0, 0), (0, 128 - linW.shape[1])))
    bp = jnp.pad(linb, (0, 128 - linb.shape[0])).reshape(1, 128)
    out = _head(ps, cnt, wp, bp, g)
    return out[:, :linW.shape[1]]
